# Initial kernel scaffold; baseline (speedup 1.0000x reference)
import jax
import jax.numpy as jnp
from jax.experimental import pallas as pl


def kernel(labels, edge_index, mask):
    raise NotImplementedError("write your pallas kernel here")


# speedup vs baseline: 5.2686x; 5.2686x over previous
import jax
import jax.numpy as jnp
from jax import lax
from jax.experimental import pallas as pl
from jax.experimental.pallas import tpu as pltpu
from jax.experimental.pallas import tpu_sc as plsc

N = 100000
E = 1600000
C = 40
ALPHA = 0.9
BETA = 1.0 - ALPHA

NC = 2
NS = 16
NG = 4
CPG = 10
W = 16

NPAD = 100352
TPN = NPAD // NS
NB = 448
NCH = TPN // NB
HALF = NPAD // 2
FPT = HALF // NS
FCH = FPT // NB

RPAD = 12544
EPAD = RPAD * 128
RT = RPAD // NS
AKK = 8
BKK = 4

_f32 = jnp.float32
_i32 = jnp.int32


def _rsqrt16(d):
    i = lax.bitcast_convert_type(d, _i32)
    i = jnp.int32(0x5F3759DF) - jnp.right_shift(i, 1)
    z = lax.bitcast_convert_type(i, _f32)
    for _ in range(3):
        z = z * (1.5 - 0.5 * d * z * z)
    return z


def _deg_body(dst2d, labels, maskv, fa, fb, fc, h0,
              deg, didx, ones, dbuf, lb, mb, fab, fbb, fcb, ob, ssem):
    cid = lax.axis_index("c")
    sid = lax.axis_index("s")
    zero16 = jnp.zeros((16,), _f32)
    one16 = jnp.ones((16,), _f32)
    iota16 = lax.iota(_i32, 16)
    for i in range(8):
        ones[pl.ds(16 * i, 16)] = one16
    for i in range(NB // 16):
        dbuf[pl.ds(16 * i, 16)] = zero16
    def _zero(u, c):
        pltpu.sync_copy(dbuf, deg.at[pl.ds(sid * TPN + u * NB, NB)])
        return c

    lax.fori_loop(0, NCH, _zero, 0)
    plsc.subcore_barrier()

    def _edge(g, carry):
        base = sid * RT + g * AKK
        pltpu.sync_copy(dst2d.at[pl.ds(base, AKK)], didx)
        descs = [pltpu.async_copy(ones, deg.at[didx.at[j]], ssem, add=True)
                 for j in range(AKK)]
        for d in descs:
            d.wait()
        return carry

    lax.fori_loop(0, RT // AKK, _edge, 0)
    plsc.subcore_barrier()

    def _fchunk(u, c):
        nb = cid * HALF + sid * FPT + u * NB
        pltpu.sync_copy(deg.at[pl.ds(nb, NB)], dbuf)

        def _fac(v, cc):
            d16 = jnp.maximum(dbuf[pl.ds(16 * v, 16)], 1.0)
            r16 = _rsqrt16(d16)
            fab[pl.ds(16 * v, 16)] = ALPHA / d16
            fbb[pl.ds(16 * v, 16)] = ALPHA * r16
            fcb[pl.ds(16 * v, 16)] = BETA * d16 * r16
            return cc

        lax.fori_loop(0, NB // 16, _fac, 0)
        pltpu.sync_copy(fab, fa.at[pl.ds(nb, NB)])
        pltpu.sync_copy(fbb, fb.at[pl.ds(nb, NB)])
        pltpu.sync_copy(fcb, fc.at[pl.ds(nb, NB)])
        return c

    lax.fori_loop(0, FCH, _fchunk, 0)

    def _hpass(gp, c):
        gabs = 2 * cid + gp

        def _hchunk(u, cc):
            nb = sid * TPN + u * NB
            pltpu.sync_copy(deg.at[pl.ds(nb, NB)], dbuf)
            pltpu.sync_copy(labels.at[pl.ds(nb, NB)], lb)
            pltpu.sync_copy(maskv.at[pl.ds(nb, NB)], mb)

            def _zrow(r, ccc):
                ob[r] = zero16
                return ccc

            lax.fori_loop(0, NB, _zrow, 0)

            def _h0(v, ccc):
                l16 = lb[pl.ds(16 * v, 16)]
                m16 = mb[pl.ds(16 * v, 16)]
                d16 = jnp.maximum(dbuf[pl.ds(16 * v, 16)], 1.0)
                r16 = _rsqrt16(d16)
                col = l16 - CPG * gabs
                valid = (m16 != 0) & (col >= 0) & (col < CPG)
                row = iota16 + 16 * v
                plsc.store_scatter(ob, [row, col], r16, mask=valid)
                return ccc

            lax.fori_loop(0, NB // 16, _h0, 0)
            pltpu.sync_copy(ob, h0.at[pl.ds(gabs * NPAD + nb, NB)])
            return cc

        lax.fori_loop(0, NCH, _hchunk, 0)
        return c

    lax.fori_loop(0, 2, _hpass, 0)


def _prop_body(src4, dst2d, h0, fa, fb, fc, zin, y4, hs,
               agg, sidx, didx, gbuf, ub, hb, f1v, f2v, gsem, ssem):
    cid = lax.axis_index("c")
    sid = lax.axis_index("s")

    for layer in range(3):
        hsrc = h0 if layer == 0 else hs
        last = layer == 2
        dref = y4 if last else hs

        def _pass(gp, carry):
            gabs = 2 * cid + gp
            pltpu.sync_copy(zin, ub)

            def _zero(u, c):
                pltpu.sync_copy(ub, agg.at[pl.ds(sid * TPN + u * NB, NB)])
                return c

            lax.fori_loop(0, NCH, _zero, 0)
            plsc.subcore_barrier()

            def _edge(g, c):
                base = sid * RT + g * BKK
                pltpu.sync_copy(dst2d.at[pl.ds(base, BKK)], didx)
                pltpu.sync_copy(src4.at[pl.ds(gabs * RPAD + base, BKK)],
                                sidx)
                gd = [pltpu.async_copy(hsrc.at[sidx.at[j]], gbuf.at[j],
                                       gsem)
                      for j in range(BKK)]
                for d in gd:
                    d.wait()
                sd = [pltpu.async_copy(gbuf.at[j], agg.at[didx.at[j]],
                                       ssem, add=True)
                      for j in range(BKK)]
                for d in sd:
                    d.wait()
                return c

            lax.fori_loop(0, RT // BKK, _edge, 0)
            plsc.subcore_barrier()

            def _chunk(u, c):
                nb = sid * TPN + u * NB
                pltpu.sync_copy(agg.at[pl.ds(nb, NB)], ub)
                pltpu.sync_copy(hsrc.at[pl.ds(gabs * NPAD + nb, NB)], hb)
                if not last:
                    pltpu.sync_copy(fa.at[pl.ds(nb, NB)], f1v)
                else:
                    pltpu.sync_copy(fb.at[pl.ds(nb, NB)], f1v)
                    pltpu.sync_copy(fc.at[pl.ds(nb, NB)], f2v)

                def _upd(i, cc):
                    f16 = f1v[pl.ds(16 * i, 16)]
                    if last:
                        g16 = f2v[pl.ds(16 * i, 16)]
                    for j in range(16):
                        r = 16 * i + j
                        av = ub[r]
                        hv = hb[r]
                        if not last:
                            res = f16[j] * av + BETA * hv
                        else:
                            res = f16[j] * av + g16[j] * hv
                        ub[r] = res
                    return cc

                lax.fori_loop(0, NB // 16, _upd, 0)
                pltpu.sync_copy(ub, dref.at[pl.ds(gabs * NPAD + nb, NB)])
                return c

            lax.fori_loop(0, NCH, _chunk, 0)
            plsc.subcore_barrier()
            return carry

        lax.fori_loop(0, 2, _pass, 0)


_MESH = plsc.VectorSubcoreMesh(core_axis_name="c", subcore_axis_name="s")
_CPARAMS = pltpu.CompilerParams(needs_layout_passes=False,
                                use_tc_tiling_on_sc=False)

_deg_kernel = pl.kernel(
    _deg_body,
    compiler_params=_CPARAMS,
    out_type=(
        jax.ShapeDtypeStruct((NPAD,), _f32),
        jax.ShapeDtypeStruct((NPAD,), _f32),
        jax.ShapeDtypeStruct((NPAD,), _f32),
        jax.ShapeDtypeStruct((NG * NPAD, W), _f32),
    ),
    mesh=_MESH,
    scratch_types=(
        pltpu.VMEM_SHARED((NPAD,), _f32),
        pltpu.VMEM((AKK, 128), _i32),
        pltpu.VMEM((128,), _f32),
        pltpu.VMEM((NB,), _f32),
        pltpu.VMEM((NB,), _i32),
        pltpu.VMEM((NB,), _i32),
        pltpu.VMEM((NB,), _f32),
        pltpu.VMEM((NB,), _f32),
        pltpu.VMEM((NB,), _f32),
        pltpu.VMEM((NB, W), _f32),
        pltpu.SemaphoreType.DMA,
    ),
)

_prop_kernel = pl.kernel(
    _prop_body,
    compiler_params=_CPARAMS,
    out_type=(
        jax.ShapeDtypeStruct((NG * NPAD, W), _f32),
        jax.ShapeDtypeStruct((NG * NPAD, W), _f32),
    ),
    mesh=_MESH,
    scratch_types=(
        pltpu.VMEM_SHARED((NPAD, W), _f32),
        pltpu.VMEM((BKK, 128), _i32),
        pltpu.VMEM((BKK, 128), _i32),
        pltpu.VMEM((BKK, 128, W), _f32),
        pltpu.VMEM((NB, W), _f32),
        pltpu.VMEM((NB, W), _f32),
        pltpu.VMEM((NB,), _f32),
        pltpu.VMEM((NB,), _f32),
        pltpu.SemaphoreType.DMA,
        pltpu.SemaphoreType.DMA,
    ),
)


def kernel(labels, edge_index, mask):
    src = edge_index[0]
    dst = edge_index[1]
    src_p = jnp.concatenate([src, jnp.zeros((EPAD - E,), _i32)])
    dst_p = jnp.concatenate([dst, jnp.full((EPAD - E,), NPAD - 1, _i32)])
    src4 = jnp.concatenate(
        [src_p + g * NPAD for g in range(NG)]).reshape(NG * RPAD, 128)
    dst2d = dst_p.reshape(RPAD, 128)
    labels_p = jnp.concatenate([labels, jnp.zeros((NPAD - N,), _i32)])
    maskv = jnp.concatenate(
        [mask.astype(_i32), jnp.zeros((NPAD - N,), _i32)])
    zin = jnp.zeros((NB, W), _f32)

    fa, fb, fc, h0 = _deg_kernel(dst2d, labels_p, maskv)
    y4, _ = _prop_kernel(src4, dst2d, h0, fa, fb, fc, zin)
    return jnp.concatenate(
        [y4[g * NPAD:g * NPAD + N, :CPG] for g in range(NG)], axis=1)


# speedup vs baseline: 7.7477x; 1.4705x over previous
import jax
import jax.numpy as jnp
from jax import lax
from jax.experimental import pallas as pl
from jax.experimental.pallas import tpu as pltpu
from jax.experimental.pallas import tpu_sc as plsc

N = 100000
E = 1600000
C = 40
ALPHA = 0.9
BETA = 1.0 - ALPHA

NC = 2
NS = 16
NG = 4
CPG = 10
W = 16

NPAD = 100352
TPN = NPAD // NS
NB = 224
NCH = TPN // NB
HALF = NPAD // 2
FPT = HALF // NS
FCH = FPT // NB

RPAD = 12544
EPAD = RPAD * 128
RT = RPAD // NS
AKK = 8
BKK = 4

_f32 = jnp.float32
_i32 = jnp.int32


def _rsqrt16(d):
    i = lax.bitcast_convert_type(d, _i32)
    i = jnp.int32(0x5F3759DF) - jnp.right_shift(i, 1)
    z = lax.bitcast_convert_type(i, _f32)
    for _ in range(3):
        z = z * (1.5 - 0.5 * d * z * z)
    return z


def _deg_body(dst2d, labels, maskv, fa, fb, fc, h0,
              deg, didx, ones, dbuf, lb, mb, fab, fbb, fcb, ob, ssem):
    cid = lax.axis_index("c")
    sid = lax.axis_index("s")
    zero16 = jnp.zeros((16,), _f32)
    one16 = jnp.ones((16,), _f32)
    iota16 = lax.iota(_i32, 16)
    for i in range(8):
        ones[pl.ds(16 * i, 16)] = one16
    for i in range(NB // 16):
        dbuf[pl.ds(16 * i, 16)] = zero16
    def _zero(u, c):
        pltpu.sync_copy(dbuf, deg.at[pl.ds(sid * TPN + u * NB, NB)])
        return c

    lax.fori_loop(0, NCH, _zero, 0)
    plsc.subcore_barrier()

    def _edge(g, carry):
        base = sid * RT + g * AKK
        pltpu.sync_copy(dst2d.at[pl.ds(base, AKK)], didx)
        descs = [pltpu.async_copy(ones, deg.at[didx.at[j]], ssem, add=True)
                 for j in range(AKK)]
        for d in descs:
            d.wait()
        return carry

    lax.fori_loop(0, RT // AKK, _edge, 0)
    plsc.subcore_barrier()

    def _fchunk(u, c):
        nb = cid * HALF + sid * FPT + u * NB
        pltpu.sync_copy(deg.at[pl.ds(nb, NB)], dbuf)

        def _fac(v, cc):
            d16 = jnp.maximum(dbuf[pl.ds(16 * v, 16)], 1.0)
            r16 = _rsqrt16(d16)
            fab[pl.ds(16 * v, 16)] = ALPHA / d16
            fbb[pl.ds(16 * v, 16)] = ALPHA * r16
            fcb[pl.ds(16 * v, 16)] = BETA * d16 * r16
            return cc

        lax.fori_loop(0, NB // 16, _fac, 0)
        pltpu.sync_copy(fab, fa.at[pl.ds(nb, NB)])
        pltpu.sync_copy(fbb, fb.at[pl.ds(nb, NB)])
        pltpu.sync_copy(fcb, fc.at[pl.ds(nb, NB)])
        return c

    lax.fori_loop(0, FCH, _fchunk, 0)

    def _hpass(gp, c):
        gabs = 2 * cid + gp

        def _hchunk(u, cc):
            nb = sid * TPN + u * NB
            pltpu.sync_copy(deg.at[pl.ds(nb, NB)], dbuf)
            pltpu.sync_copy(labels.at[pl.ds(nb, NB)], lb)
            pltpu.sync_copy(maskv.at[pl.ds(nb, NB)], mb)

            def _zrow(r, ccc):
                ob[r] = zero16
                return ccc

            lax.fori_loop(0, NB, _zrow, 0)

            def _h0(v, ccc):
                l16 = lb[pl.ds(16 * v, 16)]
                m16 = mb[pl.ds(16 * v, 16)]
                d16 = jnp.maximum(dbuf[pl.ds(16 * v, 16)], 1.0)
                r16 = _rsqrt16(d16)
                col = l16 - CPG * gabs
                valid = (m16 != 0) & (col >= 0) & (col < CPG)
                row = iota16 + 16 * v
                plsc.store_scatter(ob, [row, col], r16, mask=valid)
                return ccc

            lax.fori_loop(0, NB // 16, _h0, 0)
            pltpu.sync_copy(ob, h0.at[pl.ds(gabs * NPAD + nb, NB)])
            return cc

        lax.fori_loop(0, NCH, _hchunk, 0)
        return c

    lax.fori_loop(0, 2, _hpass, 0)


def _prop_body(src4, dst2d, h0, fa, fb, fc, zin, y4, hs,
               agg, sidx, didx, gbuf, ub, hb, f1v, f2v, gsem, ssem):
    cid = lax.axis_index("c")
    sid = lax.axis_index("s")

    for layer in range(3):
        hsrc = h0 if layer == 0 else hs
        last = layer == 2
        dref = y4 if last else hs

        def _pass(gp, carry):
            gabs = 2 * cid + gp
            pltpu.sync_copy(zin, ub)

            def _zero(u, c):
                pltpu.sync_copy(ub, agg.at[pl.ds(sid * TPN + u * NB, NB)])
                return c

            lax.fori_loop(0, NCH, _zero, 0)
            plsc.subcore_barrier()

            NBATCH = RT // BKK

            def _ldidx(b, slot):
                base = sid * RT + b * BKK
                pltpu.sync_copy(dst2d.at[pl.ds(base, BKK)],
                                didx.at[pl.ds(slot * BKK, BKK)])
                pltpu.sync_copy(src4.at[pl.ds(gabs * RPAD + base, BKK)],
                                sidx.at[pl.ds(slot * BKK, BKK)])

            def _fire_gather(slot, par):
                for j in range(BKK):
                    pltpu.async_copy(hsrc.at[sidx.at[slot * BKK + j]],
                                     gbuf.at[par * BKK + j], gsem)

            _ldidx(0, 0)
            _fire_gather(0, 0)
            _ldidx(1, 1)

            def _edge(g, c):
                p = g % 2
                s = g % 3
                for j in range(BKK):
                    pltpu.make_async_copy(
                        hsrc.at[sidx.at[s * BKK + j]],
                        gbuf.at[p * BKK + j], gsem).wait()

                @pl.when(g > 0)
                def _():
                    for j in range(BKK):
                        pltpu.make_async_copy(
                            gbuf.at[j], agg.at[didx.at[s * BKK + j]],
                            ssem).wait()

                for j in range(BKK):
                    pltpu.async_copy(gbuf.at[p * BKK + j],
                                     agg.at[didx.at[s * BKK + j]],
                                     ssem, add=True)

                @pl.when(g < NBATCH - 1)
                def _():
                    _fire_gather((g + 1) % 3, 1 - p)

                @pl.when(g < NBATCH - 2)
                def _():
                    _ldidx(g + 2, (g + 2) % 3)

                return c

            lax.fori_loop(0, NBATCH, _edge, 0)
            for j in range(BKK):
                pltpu.make_async_copy(
                    gbuf.at[j], agg.at[didx.at[j]], ssem).wait()
            plsc.subcore_barrier()

            def _chunk(u, c):
                nb = sid * TPN + u * NB
                pltpu.sync_copy(agg.at[pl.ds(nb, NB)], ub)
                pltpu.sync_copy(hsrc.at[pl.ds(gabs * NPAD + nb, NB)], hb)
                if not last:
                    pltpu.sync_copy(fa.at[pl.ds(nb, NB)], f1v)
                else:
                    pltpu.sync_copy(fb.at[pl.ds(nb, NB)], f1v)
                    pltpu.sync_copy(fc.at[pl.ds(nb, NB)], f2v)

                def _upd(i, cc):
                    f16 = f1v[pl.ds(16 * i, 16)]
                    if last:
                        g16 = f2v[pl.ds(16 * i, 16)]
                    for j in range(16):
                        r = 16 * i + j
                        av = ub[r]
                        hv = hb[r]
                        if not last:
                            res = f16[j] * av + BETA * hv
                        else:
                            res = f16[j] * av + g16[j] * hv
                        ub[r] = res
                    return cc

                lax.fori_loop(0, NB // 16, _upd, 0)
                pltpu.sync_copy(ub, dref.at[pl.ds(gabs * NPAD + nb, NB)])
                return c

            lax.fori_loop(0, NCH, _chunk, 0)
            plsc.subcore_barrier()
            return carry

        lax.fori_loop(0, 2, _pass, 0)


_MESH = plsc.VectorSubcoreMesh(core_axis_name="c", subcore_axis_name="s")
_CPARAMS = pltpu.CompilerParams(needs_layout_passes=False,
                                use_tc_tiling_on_sc=False)

_deg_kernel = pl.kernel(
    _deg_body,
    compiler_params=_CPARAMS,
    out_type=(
        jax.ShapeDtypeStruct((NPAD,), _f32),
        jax.ShapeDtypeStruct((NPAD,), _f32),
        jax.ShapeDtypeStruct((NPAD,), _f32),
        jax.ShapeDtypeStruct((NG * NPAD, W), _f32),
    ),
    mesh=_MESH,
    scratch_types=(
        pltpu.VMEM_SHARED((NPAD,), _f32),
        pltpu.VMEM((AKK, 128), _i32),
        pltpu.VMEM((128,), _f32),
        pltpu.VMEM((NB,), _f32),
        pltpu.VMEM((NB,), _i32),
        pltpu.VMEM((NB,), _i32),
        pltpu.VMEM((NB,), _f32),
        pltpu.VMEM((NB,), _f32),
        pltpu.VMEM((NB,), _f32),
        pltpu.VMEM((NB, W), _f32),
        pltpu.SemaphoreType.DMA,
    ),
)

_prop_kernel = pl.kernel(
    _prop_body,
    compiler_params=_CPARAMS,
    out_type=(
        jax.ShapeDtypeStruct((NG * NPAD, W), _f32),
        jax.ShapeDtypeStruct((NG * NPAD, W), _f32),
    ),
    mesh=_MESH,
    scratch_types=(
        pltpu.VMEM_SHARED((NPAD, W), _f32),
        pltpu.VMEM((3 * BKK, 128), _i32),
        pltpu.VMEM((3 * BKK, 128), _i32),
        pltpu.VMEM((2 * BKK, 128, W), _f32),
        pltpu.VMEM((NB, W), _f32),
        pltpu.VMEM((NB, W), _f32),
        pltpu.VMEM((NB,), _f32),
        pltpu.VMEM((NB,), _f32),
        pltpu.SemaphoreType.DMA,
        pltpu.SemaphoreType.DMA,
    ),
)


def kernel(labels, edge_index, mask):
    src = edge_index[0]
    dst = edge_index[1]
    src_p = jnp.concatenate([src, jnp.zeros((EPAD - E,), _i32)])
    dst_p = jnp.concatenate([dst, jnp.full((EPAD - E,), NPAD - 1, _i32)])
    src4 = jnp.concatenate(
        [src_p + g * NPAD for g in range(NG)]).reshape(NG * RPAD, 128)
    dst2d = dst_p.reshape(RPAD, 128)
    labels_p = jnp.concatenate([labels, jnp.zeros((NPAD - N,), _i32)])
    maskv = jnp.concatenate(
        [mask.astype(_i32), jnp.zeros((NPAD - N,), _i32)])
    zin = jnp.zeros((NB, W), _f32)

    fa, fb, fc, h0 = _deg_kernel(dst2d, labels_p, maskv)
    y4, _ = _prop_kernel(src4, dst2d, h0, fa, fb, fc, zin)
    return jnp.concatenate(
        [y4[g * NPAD:g * NPAD + N, :CPG] for g in range(NG)], axis=1)


# speedup vs baseline: 8.1302x; 1.0494x over previous
import jax
import jax.numpy as jnp
from jax import lax
from jax.experimental import pallas as pl
from jax.experimental.pallas import tpu as pltpu
from jax.experimental.pallas import tpu_sc as plsc

N = 100000
E = 1600000
C = 40
ALPHA = 0.9
BETA = 1.0 - ALPHA

NC = 2
NS = 16
NG = 4
CPG = 10
W = 16

NPAD = 100352
TPN = NPAD // NS
NB = 224
NCH = TPN // NB
HALF = NPAD // 2
FPT = HALF // NS
FCH = FPT // NB

RPAD = 12544
EPAD = RPAD * 128
RT = RPAD // NS
AKK = 8
BKK = 4

_f32 = jnp.float32
_i32 = jnp.int32


def _rsqrt16(d):
    i = lax.bitcast_convert_type(d, _i32)
    i = jnp.int32(0x5F3759DF) - jnp.right_shift(i, 1)
    z = lax.bitcast_convert_type(i, _f32)
    for _ in range(3):
        z = z * (1.5 - 0.5 * d * z * z)
    return z


def _deg_body(dst2d, labels, maskv, fa, fb, fc, h0,
              deg, didx, ones, dbuf, lb, mb, fab, fbb, fcb, ob, ssem):
    cid = lax.axis_index("c")
    sid = lax.axis_index("s")
    zero16 = jnp.zeros((16,), _f32)
    one16 = jnp.ones((16,), _f32)
    iota16 = lax.iota(_i32, 16)
    for i in range(8):
        ones[pl.ds(16 * i, 16)] = one16
    for i in range(NB // 16):
        dbuf[pl.ds(16 * i, 16)] = zero16
    def _zero(u, c):
        pltpu.sync_copy(dbuf, deg.at[pl.ds(sid * TPN + u * NB, NB)])
        return c

    lax.fori_loop(0, NCH, _zero, 0)
    plsc.subcore_barrier()

    def _edge(g, carry):
        base = sid * RT + g * AKK
        pltpu.sync_copy(dst2d.at[pl.ds(base, AKK)],
                        didx.at[pl.ds(0, AKK)])
        ds = [pltpu.async_copy(ones, deg.at[didx.at[j]], ssem, add=True)
              for j in range(AKK)]
        for d in ds:
            d.wait()
        return carry

    lax.fori_loop(0, RT // AKK, _edge, 0)
    plsc.subcore_barrier()

    def _fchunk(u, c):
        nb = cid * HALF + sid * FPT + u * NB
        pltpu.sync_copy(deg.at[pl.ds(nb, NB)], dbuf)

        def _fac(v, cc):
            d16 = jnp.maximum(dbuf[pl.ds(16 * v, 16)], 1.0)
            r16 = _rsqrt16(d16)
            fab[pl.ds(16 * v, 16)] = ALPHA / d16
            fbb[pl.ds(16 * v, 16)] = ALPHA * r16
            fcb[pl.ds(16 * v, 16)] = BETA * d16 * r16
            return cc

        lax.fori_loop(0, NB // 16, _fac, 0)
        pltpu.sync_copy(fab, fa.at[pl.ds(nb, NB)])
        pltpu.sync_copy(fbb, fb.at[pl.ds(nb, NB)])
        pltpu.sync_copy(fcb, fc.at[pl.ds(nb, NB)])
        return c

    lax.fori_loop(0, FCH, _fchunk, 0)

    def _hpass(gp, c):
        gabs = 2 * cid + gp

        def _hchunk(u, cc):
            nb = sid * TPN + u * NB
            pltpu.sync_copy(deg.at[pl.ds(nb, NB)], dbuf)
            pltpu.sync_copy(labels.at[pl.ds(nb, NB)], lb)
            pltpu.sync_copy(maskv.at[pl.ds(nb, NB)], mb)

            def _zrow(r, ccc):
                ob[r] = zero16
                return ccc

            lax.fori_loop(0, NB, _zrow, 0)

            def _h0(v, ccc):
                l16 = lb[pl.ds(16 * v, 16)]
                m16 = mb[pl.ds(16 * v, 16)]
                d16 = jnp.maximum(dbuf[pl.ds(16 * v, 16)], 1.0)
                r16 = _rsqrt16(d16)
                col = l16 - CPG * gabs
                valid = (m16 != 0) & (col >= 0) & (col < CPG)
                row = iota16 + 16 * v
                plsc.store_scatter(ob, [row, col], r16, mask=valid)
                return ccc

            lax.fori_loop(0, NB // 16, _h0, 0)
            pltpu.sync_copy(ob, h0.at[pl.ds(gabs * NPAD + nb, NB)])
            return cc

        lax.fori_loop(0, NCH, _hchunk, 0)
        return c

    lax.fori_loop(0, 2, _hpass, 0)


def _prop_body(src4, dst2d, h0, fa, fb, fc, zin, y4, hs,
               agg, sidx, didx, gbuf, ub, hb, f1v, f2v, gsem, ssem):
    cid = lax.axis_index("c")
    sid = lax.axis_index("s")

    for layer in range(3):
        hsrc = h0 if layer == 0 else hs
        last = layer == 2
        dref = y4 if last else hs

        def _pass(gp, carry):
            gabs = 2 * cid + gp
            pltpu.sync_copy(zin, ub)

            def _zero(u, c):
                pltpu.sync_copy(ub, agg.at[pl.ds(sid * TPN + u * NB, NB)])
                return c

            lax.fori_loop(0, NCH, _zero, 0)
            plsc.subcore_barrier()

            NBATCH = RT // BKK

            def _ldidx(b, slot):
                base = sid * RT + b * BKK
                pltpu.sync_copy(dst2d.at[pl.ds(base, BKK)],
                                didx.at[pl.ds(slot * BKK, BKK)])
                pltpu.sync_copy(src4.at[pl.ds(gabs * RPAD + base, BKK)],
                                sidx.at[pl.ds(slot * BKK, BKK)])

            def _fire_gather(slot, par):
                for j in range(BKK):
                    pltpu.async_copy(hsrc.at[sidx.at[slot * BKK + j]],
                                     gbuf.at[par * BKK + j], gsem)

            _ldidx(0, 0)
            _fire_gather(0, 0)
            _ldidx(1, 1)

            def _edge(g, c):
                p = g % 2
                s = g % 3
                for j in range(BKK):
                    pltpu.make_async_copy(
                        hsrc.at[sidx.at[s * BKK + j]],
                        gbuf.at[p * BKK + j], gsem).wait()

                @pl.when(g > 0)
                def _():
                    for j in range(BKK):
                        pltpu.make_async_copy(
                            gbuf.at[j], agg.at[didx.at[s * BKK + j]],
                            ssem).wait()

                for j in range(BKK):
                    pltpu.async_copy(gbuf.at[p * BKK + j],
                                     agg.at[didx.at[s * BKK + j]],
                                     ssem, add=True)

                @pl.when(g < NBATCH - 1)
                def _():
                    _fire_gather((g + 1) % 3, 1 - p)

                @pl.when(g < NBATCH - 2)
                def _():
                    _ldidx(g + 2, (g + 2) % 3)

                return c

            lax.fori_loop(0, NBATCH, _edge, 0)
            for j in range(BKK):
                pltpu.make_async_copy(
                    gbuf.at[j], agg.at[didx.at[j]], ssem).wait()
            plsc.subcore_barrier()

            def _chunk(u, c):
                nb = sid * TPN + u * NB
                cps = [(hsrc.at[pl.ds(gabs * NPAD + nb, NB)], hb)]
                if not last:
                    cps.append((fa.at[pl.ds(nb, NB)], f1v))
                else:
                    cps.append((fb.at[pl.ds(nb, NB)], f1v))
                    cps.append((fc.at[pl.ds(nb, NB)], f2v))
                ds = [pltpu.async_copy(s, d, gsem) for s, d in cps]
                pltpu.sync_copy(agg.at[pl.ds(nb, NB)], ub)
                for d in ds:
                    d.wait()

                def _upd(i, cc):
                    f16 = f1v[pl.ds(16 * i, 16)]
                    if last:
                        g16 = f2v[pl.ds(16 * i, 16)]
                    for j in range(16):
                        r = 16 * i + j
                        av = ub[r]
                        hv = hb[r]
                        if not last:
                            res = f16[j] * av + BETA * hv
                        else:
                            res = f16[j] * av + g16[j] * hv
                        ub[r] = res
                    return cc

                lax.fori_loop(0, NB // 16, _upd, 0)
                pltpu.sync_copy(ub, dref.at[pl.ds(gabs * NPAD + nb, NB)])
                return c

            lax.fori_loop(0, NCH, _chunk, 0)
            plsc.subcore_barrier()
            return carry

        lax.fori_loop(0, 2, _pass, 0)


_MESH = plsc.VectorSubcoreMesh(core_axis_name="c", subcore_axis_name="s")
_CPARAMS = pltpu.CompilerParams(needs_layout_passes=False,
                                use_tc_tiling_on_sc=False)

_deg_kernel = pl.kernel(
    _deg_body,
    compiler_params=_CPARAMS,
    out_type=(
        jax.ShapeDtypeStruct((NPAD,), _f32),
        jax.ShapeDtypeStruct((NPAD,), _f32),
        jax.ShapeDtypeStruct((NPAD,), _f32),
        jax.ShapeDtypeStruct((NG * NPAD, W), _f32),
    ),
    mesh=_MESH,
    scratch_types=(
        pltpu.VMEM_SHARED((NPAD,), _f32),
        pltpu.VMEM((2 * AKK, 128), _i32),
        pltpu.VMEM((128,), _f32),
        pltpu.VMEM((NB,), _f32),
        pltpu.VMEM((NB,), _i32),
        pltpu.VMEM((NB,), _i32),
        pltpu.VMEM((NB,), _f32),
        pltpu.VMEM((NB,), _f32),
        pltpu.VMEM((NB,), _f32),
        pltpu.VMEM((NB, W), _f32),
        pltpu.SemaphoreType.DMA,
    ),
)

_prop_kernel = pl.kernel(
    _prop_body,
    compiler_params=_CPARAMS,
    out_type=(
        jax.ShapeDtypeStruct((NG * NPAD, W), _f32),
        jax.ShapeDtypeStruct((NG * NPAD, W), _f32),
    ),
    mesh=_MESH,
    scratch_types=(
        pltpu.VMEM_SHARED((NPAD, W), _f32),
        pltpu.VMEM((3 * BKK, 128), _i32),
        pltpu.VMEM((3 * BKK, 128), _i32),
        pltpu.VMEM((2 * BKK, 128, W), _f32),
        pltpu.VMEM((NB, W), _f32),
        pltpu.VMEM((NB, W), _f32),
        pltpu.VMEM((NB,), _f32),
        pltpu.VMEM((NB,), _f32),
        pltpu.SemaphoreType.DMA,
        pltpu.SemaphoreType.DMA,
    ),
)


def kernel(labels, edge_index, mask):
    src = edge_index[0]
    dst = edge_index[1]
    src_p = jnp.concatenate([src, jnp.zeros((EPAD - E,), _i32)])
    dst_p = jnp.concatenate([dst, jnp.full((EPAD - E,), NPAD - 1, _i32)])
    src4 = jnp.concatenate(
        [src_p + g * NPAD for g in range(NG)]).reshape(NG * RPAD, 128)
    dst2d = dst_p.reshape(RPAD, 128)
    labels_p = jnp.concatenate([labels, jnp.zeros((NPAD - N,), _i32)])
    maskv = jnp.concatenate(
        [mask.astype(_i32), jnp.zeros((NPAD - N,), _i32)])
    zin = jnp.zeros((NB, W), _f32)

    fa, fb, fc, h0 = _deg_kernel(dst2d, labels_p, maskv)
    y4, _ = _prop_kernel(src4, dst2d, h0, fa, fb, fc, zin)
    return jnp.concatenate(
        [y4[g * NPAD:g * NPAD + N, :CPG] for g in range(NG)], axis=1)


# speedup vs baseline: 8.2112x; 1.0100x over previous
import jax
import jax.numpy as jnp
from jax import lax
from jax.experimental import pallas as pl
from jax.experimental.pallas import tpu as pltpu
from jax.experimental.pallas import tpu_sc as plsc

N = 100000
E = 1600000
C = 40
ALPHA = 0.9
BETA = 1.0 - ALPHA

NC = 2
NS = 16
NG = 4
CPG = 10
W = 16

NPAD = 100352
TPN = NPAD // NS
NB = 224
NCH = TPN // NB
HALF = NPAD // 2
FPT = HALF // NS
FCH = FPT // NB

RPAD = 12544
EPAD = RPAD * 128
RT = RPAD // NS
AKK = 16
BKK = 4

_f32 = jnp.float32
_i32 = jnp.int32


def _rsqrt16(d):
    i = lax.bitcast_convert_type(d, _i32)
    i = jnp.int32(0x5F3759DF) - jnp.right_shift(i, 1)
    z = lax.bitcast_convert_type(i, _f32)
    for _ in range(3):
        z = z * (1.5 - 0.5 * d * z * z)
    return z


def _deg_body(dst2d, labels, maskv, fa, fb, fc, h0,
              deg, didx, ones, dbuf, lb, mb, fab, fbb, fcb, ob, ssem):
    cid = lax.axis_index("c")
    sid = lax.axis_index("s")
    zero16 = jnp.zeros((16,), _f32)
    one16 = jnp.ones((16,), _f32)
    iota16 = lax.iota(_i32, 16)
    for i in range(8):
        ones[pl.ds(16 * i, 16)] = one16
    for i in range(NB // 16):
        dbuf[pl.ds(16 * i, 16)] = zero16
    def _zero(u, c):
        pltpu.sync_copy(dbuf, deg.at[pl.ds(sid * TPN + u * NB, NB)])
        return c

    lax.fori_loop(0, NCH, _zero, 0)
    plsc.subcore_barrier()

    def _edge(g, carry):
        base = sid * RT + g * AKK
        pltpu.sync_copy(dst2d.at[pl.ds(base, AKK)],
                        didx.at[pl.ds(0, AKK)])
        ds = [pltpu.async_copy(ones, deg.at[didx.at[j]], ssem, add=True)
              for j in range(AKK)]
        for d in ds:
            d.wait()
        return carry

    lax.fori_loop(0, RT // AKK, _edge, 0)
    plsc.subcore_barrier()

    def _fchunk(u, c):
        nb = cid * HALF + sid * FPT + u * NB
        pltpu.sync_copy(deg.at[pl.ds(nb, NB)], dbuf)

        def _fac(v, cc):
            d16 = jnp.maximum(dbuf[pl.ds(16 * v, 16)], 1.0)
            r16 = _rsqrt16(d16)
            fab[pl.ds(16 * v, 16)] = ALPHA / d16
            fbb[pl.ds(16 * v, 16)] = ALPHA * r16
            fcb[pl.ds(16 * v, 16)] = BETA * d16 * r16
            return cc

        lax.fori_loop(0, NB // 16, _fac, 0)
        pltpu.sync_copy(fab, fa.at[pl.ds(nb, NB)])
        pltpu.sync_copy(fbb, fb.at[pl.ds(nb, NB)])
        pltpu.sync_copy(fcb, fc.at[pl.ds(nb, NB)])
        return c

    lax.fori_loop(0, FCH, _fchunk, 0)

    def _hpass(gp, c):
        gabs = 2 * cid + gp

        def _hchunk(u, cc):
            nb = sid * TPN + u * NB
            pltpu.sync_copy(deg.at[pl.ds(nb, NB)], dbuf)
            pltpu.sync_copy(labels.at[pl.ds(nb, NB)], lb)
            pltpu.sync_copy(maskv.at[pl.ds(nb, NB)], mb)

            def _zrow(r, ccc):
                ob[r] = zero16
                return ccc

            lax.fori_loop(0, NB, _zrow, 0)

            def _h0(v, ccc):
                l16 = lb[pl.ds(16 * v, 16)]
                m16 = mb[pl.ds(16 * v, 16)]
                d16 = jnp.maximum(dbuf[pl.ds(16 * v, 16)], 1.0)
                r16 = _rsqrt16(d16)
                col = l16 - CPG * gabs
                valid = (m16 != 0) & (col >= 0) & (col < CPG)
                row = iota16 + 16 * v
                plsc.store_scatter(ob, [row, col], r16, mask=valid)
                return ccc

            lax.fori_loop(0, NB // 16, _h0, 0)
            pltpu.sync_copy(ob, h0.at[pl.ds(gabs * NPAD + nb, NB)])
            return cc

        lax.fori_loop(0, NCH, _hchunk, 0)
        return c

    lax.fori_loop(0, 2, _hpass, 0)


def _prop_body(src4, dst2d, h0, fa, fb, fc, zin, y4, hs,
               agg, sidx, didx, gbuf, ub, hb, f1v, f2v, gsem, ssem):
    cid = lax.axis_index("c")
    sid = lax.axis_index("s")

    for layer in range(3):
        hsrc = h0 if layer == 0 else hs
        last = layer == 2
        dref = y4 if last else hs

        def _pass(gp, carry):
            gabs = 2 * cid + gp
            pltpu.sync_copy(zin, ub)

            def _zero(u, c):
                pltpu.sync_copy(ub, agg.at[pl.ds(sid * TPN + u * NB, NB)])
                return c

            lax.fori_loop(0, NCH, _zero, 0)
            plsc.subcore_barrier()

            NBATCH = RT // BKK

            def _ldidx(b, slot):
                base = sid * RT + b * BKK
                pltpu.sync_copy(dst2d.at[pl.ds(base, BKK)],
                                didx.at[pl.ds(slot * BKK, BKK)])
                pltpu.sync_copy(src4.at[pl.ds(gabs * RPAD + base, BKK)],
                                sidx.at[pl.ds(slot * BKK, BKK)])

            def _fire_gather(slot, par):
                for j in range(BKK):
                    pltpu.async_copy(hsrc.at[sidx.at[slot * BKK + j]],
                                     gbuf.at[par * BKK + j], gsem)

            _ldidx(0, 0)
            _fire_gather(0, 0)
            _ldidx(1, 1)

            def _edge(g, c):
                p = g % 2
                s = g % 3
                for j in range(BKK):
                    pltpu.make_async_copy(
                        hsrc.at[sidx.at[s * BKK + j]],
                        gbuf.at[p * BKK + j], gsem).wait()

                @pl.when(g > 0)
                def _():
                    for j in range(BKK):
                        pltpu.make_async_copy(
                            gbuf.at[j], agg.at[didx.at[s * BKK + j]],
                            ssem).wait()

                for j in range(BKK):
                    pltpu.async_copy(gbuf.at[p * BKK + j],
                                     agg.at[didx.at[s * BKK + j]],
                                     ssem, add=True)

                @pl.when(g < NBATCH - 1)
                def _():
                    _fire_gather((g + 1) % 3, 1 - p)

                @pl.when(g < NBATCH - 2)
                def _():
                    _ldidx(g + 2, (g + 2) % 3)

                return c

            lax.fori_loop(0, NBATCH, _edge, 0)
            for j in range(BKK):
                pltpu.make_async_copy(
                    gbuf.at[j], agg.at[didx.at[j]], ssem).wait()
            plsc.subcore_barrier()

            def _chunk(u, c):
                nb = sid * TPN + u * NB
                cps = [(hsrc.at[pl.ds(gabs * NPAD + nb, NB)], hb)]
                if not last:
                    cps.append((fa.at[pl.ds(nb, NB)], f1v))
                else:
                    cps.append((fb.at[pl.ds(nb, NB)], f1v))
                    cps.append((fc.at[pl.ds(nb, NB)], f2v))
                ds = [pltpu.async_copy(s, d, gsem) for s, d in cps]
                pltpu.sync_copy(agg.at[pl.ds(nb, NB)], ub)
                for d in ds:
                    d.wait()

                def _upd(i, cc):
                    f16 = f1v[pl.ds(16 * i, 16)]
                    if last:
                        g16 = f2v[pl.ds(16 * i, 16)]
                    for j in range(16):
                        r = 16 * i + j
                        av = ub[r]
                        hv = hb[r]
                        if not last:
                            res = f16[j] * av + BETA * hv
                        else:
                            res = f16[j] * av + g16[j] * hv
                        ub[r] = res
                    return cc

                lax.fori_loop(0, NB // 16, _upd, 0)
                pltpu.sync_copy(ub, dref.at[pl.ds(gabs * NPAD + nb, NB)])
                return c

            lax.fori_loop(0, NCH, _chunk, 0)
            plsc.subcore_barrier()
            return carry

        lax.fori_loop(0, 2, _pass, 0)


_MESH = plsc.VectorSubcoreMesh(core_axis_name="c", subcore_axis_name="s")
_CPARAMS = pltpu.CompilerParams(needs_layout_passes=False,
                                use_tc_tiling_on_sc=False)

_deg_kernel = pl.kernel(
    _deg_body,
    compiler_params=_CPARAMS,
    out_type=(
        jax.ShapeDtypeStruct((NPAD,), _f32),
        jax.ShapeDtypeStruct((NPAD,), _f32),
        jax.ShapeDtypeStruct((NPAD,), _f32),
        jax.ShapeDtypeStruct((NG * NPAD, W), _f32),
    ),
    mesh=_MESH,
    scratch_types=(
        pltpu.VMEM_SHARED((NPAD,), _f32),
        pltpu.VMEM((AKK, 128), _i32),
        pltpu.VMEM((128,), _f32),
        pltpu.VMEM((NB,), _f32),
        pltpu.VMEM((NB,), _i32),
        pltpu.VMEM((NB,), _i32),
        pltpu.VMEM((NB,), _f32),
        pltpu.VMEM((NB,), _f32),
        pltpu.VMEM((NB,), _f32),
        pltpu.VMEM((NB, W), _f32),
        pltpu.SemaphoreType.DMA,
    ),
)

_prop_kernel = pl.kernel(
    _prop_body,
    compiler_params=_CPARAMS,
    out_type=(
        jax.ShapeDtypeStruct((NG * NPAD, W), _f32),
        jax.ShapeDtypeStruct((NG * NPAD, W), _f32),
    ),
    mesh=_MESH,
    scratch_types=(
        pltpu.VMEM_SHARED((NPAD, W), _f32),
        pltpu.VMEM((3 * BKK, 128), _i32),
        pltpu.VMEM((3 * BKK, 128), _i32),
        pltpu.VMEM((2 * BKK, 128, W), _f32),
        pltpu.VMEM((NB, W), _f32),
        pltpu.VMEM((NB, W), _f32),
        pltpu.VMEM((NB,), _f32),
        pltpu.VMEM((NB,), _f32),
        pltpu.SemaphoreType.DMA,
        pltpu.SemaphoreType.DMA,
    ),
)


def kernel(labels, edge_index, mask):
    src = edge_index[0]
    dst = edge_index[1]
    src_p = jnp.concatenate([src, jnp.zeros((EPAD - E,), _i32)])
    dst_p = jnp.concatenate([dst, jnp.full((EPAD - E,), NPAD - 1, _i32)])
    src4 = jnp.concatenate(
        [src_p + g * NPAD for g in range(NG)]).reshape(NG * RPAD, 128)
    dst2d = dst_p.reshape(RPAD, 128)
    labels_p = jnp.concatenate([labels, jnp.zeros((NPAD - N,), _i32)])
    maskv = jnp.concatenate(
        [mask.astype(_i32), jnp.zeros((NPAD - N,), _i32)])
    zin = jnp.zeros((NB, W), _f32)

    fa, fb, fc, h0 = _deg_kernel(dst2d, labels_p, maskv)
    y4, _ = _prop_kernel(src4, dst2d, h0, fa, fb, fc, zin)
    return jnp.concatenate(
        [y4[g * NPAD:g * NPAD + N, :CPG] for g in range(NG)], axis=1)


# speedup vs baseline: 8.2738x; 1.0076x over previous
import jax
import jax.numpy as jnp
from jax import lax
from jax.experimental import pallas as pl
from jax.experimental.pallas import tpu as pltpu
from jax.experimental.pallas import tpu_sc as plsc

N = 100000
E = 1600000
C = 40
ALPHA = 0.9
BETA = 1.0 - ALPHA

NC = 2
NS = 16
NG = 4
CPG = 10
W = 16

NPAD = 100352
TPN = NPAD // NS
NB = 224
NCH = TPN // NB
HALF = NPAD // 2
FPT = HALF // NS
FCH = FPT // NB

RPAD = 12544
EPAD = RPAD * 128
RT = RPAD // NS
AKK = 16
BKK = 4

_f32 = jnp.float32
_i32 = jnp.int32


def _rsqrt16(d):
    i = lax.bitcast_convert_type(d, _i32)
    i = jnp.int32(0x5F3759DF) - jnp.right_shift(i, 1)
    z = lax.bitcast_convert_type(i, _f32)
    for _ in range(3):
        z = z * (1.5 - 0.5 * d * z * z)
    return z


def _deg_body(dst2d, labels, maskv, fa, fb, fc, h0,
              deg, didx, ones, dbuf, lb, mb, fab, fbb, fcb, ob, ssem):
    cid = lax.axis_index("c")
    sid = lax.axis_index("s")
    zero16 = jnp.zeros((16,), _f32)
    one16 = jnp.ones((16,), _f32)
    iota16 = lax.iota(_i32, 16)
    for i in range(8):
        ones[pl.ds(16 * i, 16)] = one16
    for i in range(NB // 16):
        dbuf[pl.ds(16 * i, 16)] = zero16
    def _zero(u, c):
        pltpu.sync_copy(dbuf, deg.at[pl.ds(sid * TPN + u * NB, NB)])
        return c

    lax.fori_loop(0, NCH, _zero, 0)
    plsc.subcore_barrier()

    def _edge(g, carry):
        base = sid * RT + g * AKK
        pltpu.sync_copy(dst2d.at[pl.ds(base, AKK)],
                        didx.at[pl.ds(0, AKK)])
        ds = [pltpu.async_copy(ones, deg.at[didx.at[j]], ssem, add=True)
              for j in range(AKK)]
        for d in ds:
            d.wait()
        return carry

    lax.fori_loop(0, RT // AKK, _edge, 0)
    plsc.subcore_barrier()

    def _fchunk(u, c):
        nb = cid * HALF + sid * FPT + u * NB
        pltpu.sync_copy(deg.at[pl.ds(nb, NB)], dbuf)

        def _fac(v, cc):
            d16 = jnp.maximum(dbuf[pl.ds(16 * v, 16)], 1.0)
            r16 = _rsqrt16(d16)
            fab[pl.ds(16 * v, 16)] = ALPHA / d16
            fbb[pl.ds(16 * v, 16)] = ALPHA * r16
            fcb[pl.ds(16 * v, 16)] = BETA * d16 * r16
            return cc

        lax.fori_loop(0, NB // 16, _fac, 0)
        pltpu.sync_copy(fab, fa.at[pl.ds(nb, NB)])
        pltpu.sync_copy(fbb, fb.at[pl.ds(nb, NB)])
        pltpu.sync_copy(fcb, fc.at[pl.ds(nb, NB)])
        return c

    lax.fori_loop(0, FCH, _fchunk, 0)

    def _hpass(gp, c):
        gabs = 2 * cid + gp

        def _hchunk(u, cc):
            nb = sid * TPN + u * NB
            pltpu.sync_copy(deg.at[pl.ds(nb, NB)], dbuf)
            pltpu.sync_copy(labels.at[pl.ds(nb, NB)], lb)
            pltpu.sync_copy(maskv.at[pl.ds(nb, NB)], mb)

            def _zrow(r, ccc):
                ob[r] = zero16
                return ccc

            lax.fori_loop(0, NB, _zrow, 0)

            def _h0(v, ccc):
                l16 = lb[pl.ds(16 * v, 16)]
                m16 = mb[pl.ds(16 * v, 16)]
                d16 = jnp.maximum(dbuf[pl.ds(16 * v, 16)], 1.0)
                r16 = _rsqrt16(d16)
                col = l16 - CPG * gabs
                valid = (m16 != 0) & (col >= 0) & (col < CPG)
                row = iota16 + 16 * v
                plsc.store_scatter(ob, [row, col], r16, mask=valid)
                return ccc

            lax.fori_loop(0, NB // 16, _h0, 0)
            pltpu.sync_copy(ob, h0.at[pl.ds(gabs * NPAD + nb, NB)])
            return cc

        lax.fori_loop(0, NCH, _hchunk, 0)
        return c

    lax.fori_loop(0, 2, _hpass, 0)


def _prop_body(src4, dst2d, h0, fa, fb, fc, zin, y4, hs,
               agg, sidx, didx, gbuf, ub, hb, f1v, f2v, gsem, ssem):
    cid = lax.axis_index("c")
    sid = lax.axis_index("s")

    for layer in range(3):
        hsrc = h0 if layer == 0 else hs
        last = layer == 2
        dref = y4 if last else hs

        def _pass(gp, carry):
            gabs = 2 * cid + gp
            pltpu.sync_copy(zin, ub)

            def _zero(u, c):
                pltpu.sync_copy(ub, agg.at[pl.ds(sid * TPN + u * NB, NB)])
                return c

            lax.fori_loop(0, NCH, _zero, 0)
            plsc.subcore_barrier()

            NBATCH = RT // BKK

            def _ldidx(b, slot):
                base = sid * RT + b * BKK
                pltpu.sync_copy(dst2d.at[pl.ds(base, BKK)],
                                didx.at[pl.ds(slot * BKK, BKK)])
                pltpu.sync_copy(src4.at[pl.ds(gabs * RPAD + base, BKK)],
                                sidx.at[pl.ds(slot * BKK, BKK)])

            def _fire_gather(slot, par):
                for j in range(BKK):
                    pltpu.async_copy(hsrc.at[sidx.at[slot * BKK + j]],
                                     gbuf.at[par * BKK + j], gsem)

            _ldidx(0, 0)
            _fire_gather(0, 0)
            _ldidx(1, 1)

            def _edge(g, c):
                p = g % 2
                s = g % 3

                @pl.when(g > 0)
                def _():
                    for j in range(BKK):
                        pltpu.make_async_copy(
                            gbuf.at[j], agg.at[didx.at[s * BKK + j]],
                            ssem).wait()

                for j in range(BKK):
                    pltpu.make_async_copy(
                        hsrc.at[sidx.at[s * BKK + j]],
                        gbuf.at[p * BKK + j], gsem).wait()
                    pltpu.async_copy(gbuf.at[p * BKK + j],
                                     agg.at[didx.at[s * BKK + j]],
                                     ssem, add=True)

                @pl.when(g < NBATCH - 1)
                def _():
                    _fire_gather((g + 1) % 3, 1 - p)

                @pl.when(g < NBATCH - 2)
                def _():
                    _ldidx(g + 2, (g + 2) % 3)

                return c

            lax.fori_loop(0, NBATCH, _edge, 0)
            for j in range(BKK):
                pltpu.make_async_copy(
                    gbuf.at[j], agg.at[didx.at[j]], ssem).wait()
            plsc.subcore_barrier()

            def _chunk(u, c):
                nb = sid * TPN + u * NB
                cps = [(hsrc.at[pl.ds(gabs * NPAD + nb, NB)], hb)]
                if not last:
                    cps.append((fa.at[pl.ds(nb, NB)], f1v))
                else:
                    cps.append((fb.at[pl.ds(nb, NB)], f1v))
                    cps.append((fc.at[pl.ds(nb, NB)], f2v))
                ds = [pltpu.async_copy(s, d, gsem) for s, d in cps]
                pltpu.sync_copy(agg.at[pl.ds(nb, NB)], ub)
                for d in ds:
                    d.wait()

                def _upd(i, cc):
                    f16 = f1v[pl.ds(16 * i, 16)]
                    if last:
                        g16 = f2v[pl.ds(16 * i, 16)]
                    for j in range(16):
                        r = 16 * i + j
                        av = ub[r]
                        hv = hb[r]
                        if not last:
                            res = f16[j] * av + BETA * hv
                        else:
                            res = f16[j] * av + g16[j] * hv
                        ub[r] = res
                    return cc

                lax.fori_loop(0, NB // 16, _upd, 0)
                pltpu.sync_copy(ub, dref.at[pl.ds(gabs * NPAD + nb, NB)])
                return c

            lax.fori_loop(0, NCH, _chunk, 0)
            plsc.subcore_barrier()
            return carry

        lax.fori_loop(0, 2, _pass, 0)


_MESH = plsc.VectorSubcoreMesh(core_axis_name="c", subcore_axis_name="s")
_CPARAMS = pltpu.CompilerParams(needs_layout_passes=False,
                                use_tc_tiling_on_sc=False)

_deg_kernel = pl.kernel(
    _deg_body,
    compiler_params=_CPARAMS,
    out_type=(
        jax.ShapeDtypeStruct((NPAD,), _f32),
        jax.ShapeDtypeStruct((NPAD,), _f32),
        jax.ShapeDtypeStruct((NPAD,), _f32),
        jax.ShapeDtypeStruct((NG * NPAD, W), _f32),
    ),
    mesh=_MESH,
    scratch_types=(
        pltpu.VMEM_SHARED((NPAD,), _f32),
        pltpu.VMEM((AKK, 128), _i32),
        pltpu.VMEM((128,), _f32),
        pltpu.VMEM((NB,), _f32),
        pltpu.VMEM((NB,), _i32),
        pltpu.VMEM((NB,), _i32),
        pltpu.VMEM((NB,), _f32),
        pltpu.VMEM((NB,), _f32),
        pltpu.VMEM((NB,), _f32),
        pltpu.VMEM((NB, W), _f32),
        pltpu.SemaphoreType.DMA,
    ),
)

_prop_kernel = pl.kernel(
    _prop_body,
    compiler_params=_CPARAMS,
    out_type=(
        jax.ShapeDtypeStruct((NG * NPAD, W), _f32),
        jax.ShapeDtypeStruct((NG * NPAD, W), _f32),
    ),
    mesh=_MESH,
    scratch_types=(
        pltpu.VMEM_SHARED((NPAD, W), _f32),
        pltpu.VMEM((3 * BKK, 128), _i32),
        pltpu.VMEM((3 * BKK, 128), _i32),
        pltpu.VMEM((2 * BKK, 128, W), _f32),
        pltpu.VMEM((NB, W), _f32),
        pltpu.VMEM((NB, W), _f32),
        pltpu.VMEM((NB,), _f32),
        pltpu.VMEM((NB,), _f32),
        pltpu.SemaphoreType.DMA,
        pltpu.SemaphoreType.DMA,
    ),
)


def kernel(labels, edge_index, mask):
    src = edge_index[0]
    dst = edge_index[1]
    src_p = jnp.concatenate([src, jnp.zeros((EPAD - E,), _i32)])
    dst_p = jnp.concatenate([dst, jnp.full((EPAD - E,), NPAD - 1, _i32)])
    src4 = jnp.concatenate(
        [src_p + g * NPAD for g in range(NG)]).reshape(NG * RPAD, 128)
    dst2d = dst_p.reshape(RPAD, 128)
    labels_p = jnp.concatenate([labels, jnp.zeros((NPAD - N,), _i32)])
    maskv = jnp.concatenate(
        [mask.astype(_i32), jnp.zeros((NPAD - N,), _i32)])
    zin = jnp.zeros((NB, W), _f32)

    fa, fb, fc, h0 = _deg_kernel(dst2d, labels_p, maskv)
    y4, _ = _prop_kernel(src4, dst2d, h0, fa, fb, fc, zin)
    return jnp.concatenate(
        [y4[g * NPAD:g * NPAD + N, :CPG] for g in range(NG)], axis=1)
